# Initial kernel scaffold; baseline (speedup 1.0000x reference)
import functools

import jax
import jax.numpy as jnp
from jax import lax
from jax.experimental import pallas as pl
from jax.experimental.pallas import tpu as pltpu
from jax.experimental.pallas import tpu_sc as plsc

N = 10000
D = 128
T = 144
NP = 10016
EPS = 1e-5
NC = 2
NS = 16
NW = NC * NS
CHUNK = 128
RPT = NP // NS
BS = 1252
GRID = NP // BS


def _logmap0_cols(x, sqrt_c):
    nrm = jnp.maximum(jnp.sqrt(jnp.sum(x * x, axis=1, keepdims=True)), EPS)
    arg = jnp.clip(sqrt_c * nrm, -1.0 + EPS, 1.0 - EPS)
    at = 0.5 * jnp.log((1.0 + arg) / (1.0 - arg))
    return at * x / (sqrt_c * nrm)


def _expmap0_rows(v, sqrt_c):
    nrm = jnp.maximum(jnp.sqrt(jnp.sum(v * v, axis=1, keepdims=True)), EPS)
    return jnp.tanh(sqrt_c * nrm) * v / (sqrt_c * nrm)


def _table_kernel(x_ref, c_ref, o_ref):
    sqrt_c = jnp.sqrt(jnp.abs(c_ref[0, 0]))
    tan = _logmap0_cols(x_ref[...], sqrt_c)
    o_ref[:, :D] = tan
    o_ref[:, D:] = jnp.concatenate(
        [jnp.ones((BS, 1), jnp.float32), jnp.zeros((BS, T - D - 1), jnp.float32)],
        axis=1,
    )


def _make_table(x_pad, c_arr, interpret=False):
    return pl.pallas_call(
        _table_kernel,
        grid=(GRID,),
        in_specs=[
            pl.BlockSpec((BS, D), lambda i: (i, 0)),
            pl.BlockSpec((1, 1), lambda i: (0, 0)),
        ],
        out_specs=pl.BlockSpec((BS, T), lambda i: (i, 0)),
        out_shape=jax.ShapeDtypeStruct((NP, T), jnp.float32),
        interpret=interpret,
    )(x_pad, c_arr)


def _combine_kernel(last, acc_ref, w_ref, b_ref, c_ref, o_ref):
    c = jnp.abs(c_ref[0, 0])
    sqrt_c = jnp.sqrt(c)
    s = acc_ref[0] + acc_ref[1]
    agg = s[:, :D]
    deg = jnp.sum(s[:, D:], axis=1, keepdims=True)
    degc = jnp.maximum(deg, 1.0)
    has = (deg >= 0.5).astype(jnp.float32)
    neigh = (
        jnp.dot(agg / degc, w_ref[...], preferred_element_type=jnp.float32)
        + has * b_ref[...]
    )
    x2 = _expmap0_rows(neigh, sqrt_c)
    if last:
        o_ref[...] = x2
    else:
        o_ref[:, :D] = _logmap0_cols(x2, sqrt_c)
        o_ref[:, D:] = jnp.concatenate(
            [jnp.ones((BS, 1), jnp.float32), jnp.zeros((BS, T - D - 1), jnp.float32)],
            axis=1,
        )


def _combine(acc, wt, b_row, c_arr, last, interpret=False):
    ow = D if last else T
    return pl.pallas_call(
        functools.partial(_combine_kernel, last),
        grid=(GRID,),
        in_specs=[
            pl.BlockSpec((2, BS, T), lambda i: (0, i, 0)),
            pl.BlockSpec((D, D), lambda i: (0, 0)),
            pl.BlockSpec((1, D), lambda i: (0, 0)),
            pl.BlockSpec((1, 1), lambda i: (0, 0)),
        ],
        out_specs=pl.BlockSpec((BS, ow), lambda i: (i, 0)),
        out_shape=jax.ShapeDtypeStruct((NP, ow), jnp.float32),
        interpret=interpret,
    )(acc, wt, b_row, c_arr)


def _sc_aggregate(table, src3, dst3, zeros, gpw):
    mesh = plsc.VectorSubcoreMesh(core_axis_name="c", subcore_axis_name="s")

    @functools.partial(
        pl.kernel,
        out_type=jax.ShapeDtypeStruct((NC, NP, T), jnp.float32),
        mesh=mesh,
        scratch_types=[
            pltpu.VMEM((gpw, CHUNK), jnp.int32),
            pltpu.VMEM((gpw, CHUNK), jnp.int32),
            pltpu.VMEM((CHUNK, T), jnp.float32),
            pltpu.VMEM_SHARED((NP, T), jnp.float32),
            pltpu.SemaphoreType.DMA,
        ],
    )
    def k(table_hbm, src_hbm, dst_hbm, zeros_hbm, out_hbm,
          src_v, dst_v, rows_v, acc_sh, sem):
        cid = lax.axis_index("c")
        sid = lax.axis_index("s")
        wid = cid * NS + sid
        pltpu.sync_copy(zeros_hbm.at[pl.ds(sid * RPT, RPT)],
                        acc_sh.at[pl.ds(sid * RPT, RPT)])
        pltpu.sync_copy(src_hbm.at[wid], src_v)
        pltpu.sync_copy(dst_hbm.at[wid], dst_v)
        plsc.subcore_barrier()

        def step(g, carry):
            pltpu.async_copy(table_hbm.at[src_v.at[g]], rows_v, sem).wait()
            pltpu.sync_copy(rows_v, acc_sh.at[dst_v.at[g]], add=True)
            return carry

        lax.fori_loop(0, gpw, step, 0)
        plsc.subcore_barrier()
        pltpu.sync_copy(acc_sh.at[pl.ds(sid * RPT, RPT)],
                        out_hbm.at[cid].at[pl.ds(sid * RPT, RPT)])

    return k(table, src3, dst3, zeros)


def kernel(node_embeddings, W1, b1, W2, b2, curvature, edge_index):
    E = edge_index.shape[1]
    gpw = -(-E // (NW * CHUNK))
    ep = NW * gpw * CHUNK
    ei = edge_index.astype(jnp.int32)
    pad = ep - E
    src3 = jnp.concatenate([ei[0], jnp.zeros((pad,), jnp.int32)]).reshape(
        NW, gpw, CHUNK)
    dst3 = jnp.concatenate([ei[1], jnp.full((pad,), NP - 1, jnp.int32)]).reshape(
        NW, gpw, CHUNK)
    x_pad = jnp.pad(node_embeddings, ((0, NP - N), (0, 0)))
    c_arr = jnp.reshape(curvature, (1, 1)).astype(jnp.float32)
    zeros = jnp.zeros((NP, T), jnp.float32)
    w1t = W1.T
    w2t = W2.T
    b1r = jnp.reshape(b1, (1, D))
    b2r = jnp.reshape(b2, (1, D))

    table1 = _make_table(x_pad, c_arr)
    acc1 = _sc_aggregate(table1, src3, dst3, zeros, gpw)
    table2 = _combine(acc1, w1t, b1r, c_arr, last=False)
    acc2 = _sc_aggregate(table2, src3, dst3, zeros, gpw)
    out = _combine(acc2, w2t, b2r, c_arr, last=True)
    return out[:N]


# speedup vs baseline: 3.8595x; 3.8595x over previous
import functools

import jax
import jax.numpy as jnp
from jax import lax
from jax.experimental import pallas as pl
from jax.experimental.pallas import tpu as pltpu
from jax.experimental.pallas import tpu_sc as plsc

N = 10000
D = 128
NP = 10240
EPS = 1e-5
NC = 2
NS = 16
NW = NC * NS
CHUNK = 128
RPT = NP // NS
HR = NP // 128
HPT = HR // NS
BS = 1280
GRID = NP // BS


def _logmap0_rows(x, sqrt_c):
    nrm = jnp.maximum(jnp.sqrt(jnp.sum(x * x, axis=1, keepdims=True)), EPS)
    arg = jnp.clip(sqrt_c * nrm, -1.0 + EPS, 1.0 - EPS)
    at = 0.5 * jnp.log((1.0 + arg) / (1.0 - arg))
    return at * x / (sqrt_c * nrm)


def _expmap0_rows(v, sqrt_c):
    nrm = jnp.maximum(jnp.sqrt(jnp.sum(v * v, axis=1, keepdims=True)), EPS)
    return jnp.tanh(sqrt_c * nrm) * v / (sqrt_c * nrm)


def _table_kernel(x_ref, c_ref, o_ref):
    sqrt_c = jnp.sqrt(jnp.abs(c_ref[0, 0]))
    o_ref[...] = _logmap0_rows(x_ref[...], sqrt_c)


def _make_table(x_pad, c_arr, interpret=False):
    return pl.pallas_call(
        _table_kernel,
        grid=(GRID,),
        in_specs=[
            pl.BlockSpec((BS, D), lambda i: (i, 0)),
            pl.BlockSpec((1, 1), lambda i: (0, 0)),
        ],
        out_specs=pl.BlockSpec((BS, D), lambda i: (i, 0)),
        out_shape=jax.ShapeDtypeStruct((NP, D), jnp.float32),
        interpret=interpret,
    )(x_pad, c_arr)


def _combine_kernel(last, acc_ref, deg_ref, w_ref, b_ref, c_ref, o_ref):
    sqrt_c = jnp.sqrt(jnp.abs(c_ref[0, 0]))
    agg = acc_ref[0] + acc_ref[1]
    deg = jnp.sum(deg_ref[...], axis=0)
    degc = jnp.maximum(deg, 1.0)
    has = (deg >= 0.5).astype(jnp.float32)
    neigh = (
        jnp.dot(agg / degc, w_ref[...], preferred_element_type=jnp.float32)
        + has * b_ref[...]
    )
    x2 = _expmap0_rows(neigh, sqrt_c)
    if last:
        o_ref[...] = x2
    else:
        o_ref[...] = _logmap0_rows(x2, sqrt_c)


def _combine(acc, deg3, wt, b_row, c_arr, last, interpret=False):
    return pl.pallas_call(
        functools.partial(_combine_kernel, last),
        grid=(GRID,),
        in_specs=[
            pl.BlockSpec((2, BS, D), lambda i: (0, i, 0)),
            pl.BlockSpec((NW, BS, 1), lambda i: (0, i, 0)),
            pl.BlockSpec((D, D), lambda i: (0, 0)),
            pl.BlockSpec((1, D), lambda i: (0, 0)),
            pl.BlockSpec((1, 1), lambda i: (0, 0)),
        ],
        out_specs=pl.BlockSpec((BS, D), lambda i: (i, 0)),
        out_shape=jax.ShapeDtypeStruct((NP, D), jnp.float32),
        interpret=interpret,
    )(acc, deg3, wt, b_row, c_arr)


def _sc_aggregate(table, src3, dst3, zeros, gpw, with_deg):
    mesh = plsc.VectorSubcoreMesh(core_axis_name="c", subcore_axis_name="s")

    acc_t = jax.ShapeDtypeStruct((NC, NP, D), jnp.float32)
    deg_t = jax.ShapeDtypeStruct((NC, NS, NP), jnp.float32)
    out_type = (acc_t, deg_t) if with_deg else acc_t
    scratch = [
        pltpu.VMEM((gpw, CHUNK), jnp.int32),
        pltpu.VMEM((gpw, CHUNK), jnp.int32),
        pltpu.VMEM((CHUNK, D), jnp.float32),
        pltpu.VMEM_SHARED((NP, D), jnp.float32),
        pltpu.SemaphoreType.DMA,
    ]
    if with_deg:
        scratch += [
            pltpu.VMEM((NP,), jnp.float32),
        ]

    @functools.partial(pl.kernel, out_type=out_type, mesh=mesh,
                       scratch_types=scratch,
                       compiler_params=pltpu.CompilerParams(
                           needs_layout_passes=False))
    def k(table_hbm, src_hbm, dst_hbm, zeros_hbm, *rest):
        if with_deg:
            (acc_out, deg_out, src_v, dst_v, rows_v, acc_sh, sem,
             hist_v) = rest
        else:
            (acc_out, src_v, dst_v, rows_v, acc_sh, sem) = rest
        cid = lax.axis_index("c")
        sid = lax.axis_index("s")
        wid = cid * NS + sid
        pltpu.sync_copy(zeros_hbm.at[pl.ds(sid * RPT, RPT)],
                        acc_sh.at[pl.ds(sid * RPT, RPT)])
        pltpu.sync_copy(src_hbm.at[wid], src_v)
        pltpu.sync_copy(dst_hbm.at[wid], dst_v)
        if with_deg:
            zeros16 = jnp.zeros((16,), jnp.float32)

            def zstep(j, carry):
                hist_v[pl.ds(j * 16, 16)] = zeros16
                return carry
            lax.fori_loop(0, NP // 16, zstep, 0)
        plsc.subcore_barrier()

        ones16 = jnp.ones((16,), jnp.float32)

        def step(g, carry):
            pltpu.async_copy(table_hbm.at[src_v.at[g]], rows_v, sem).wait()
            pltpu.sync_copy(rows_v, acc_sh.at[dst_v.at[g]], add=True)
            if with_deg:
                def hstep(j, c2):
                    v = dst_v[g, pl.ds(j * 16, 16)]
                    plsc.addupdate_scatter(hist_v, [v], ones16)
                    return c2
                lax.fori_loop(0, CHUNK // 16, hstep, 0)
            return carry

        lax.fori_loop(0, gpw, step, 0)
        if with_deg:
            pltpu.sync_copy(hist_v, deg_out.at[cid].at[sid])
        plsc.subcore_barrier()
        pltpu.sync_copy(acc_sh.at[pl.ds(sid * RPT, RPT)],
                        acc_out.at[cid].at[pl.ds(sid * RPT, RPT)])

    return k(table, src3, dst3, zeros)


def kernel(node_embeddings, W1, b1, W2, b2, curvature, edge_index):
    E = edge_index.shape[1]
    gpw = -(-E // (NW * CHUNK))
    ep = NW * gpw * CHUNK
    ei = edge_index.astype(jnp.int32)
    pad = ep - E
    src3 = jnp.concatenate([ei[0], jnp.zeros((pad,), jnp.int32)]).reshape(
        NW, gpw, CHUNK)
    dst3 = jnp.concatenate([ei[1], jnp.full((pad,), NP - 1, jnp.int32)]).reshape(
        NW, gpw, CHUNK)
    x_pad = jnp.pad(node_embeddings, ((0, NP - N), (0, 0)))
    c_arr = jnp.reshape(curvature, (1, 1)).astype(jnp.float32)
    zeros = jnp.zeros((NP, D), jnp.float32)
    w1t = W1.T
    w2t = W2.T
    b1r = jnp.reshape(b1, (1, D))
    b2r = jnp.reshape(b2, (1, D))

    table1 = _make_table(x_pad, c_arr)
    acc1, deg = _sc_aggregate(table1, src3, dst3, zeros, gpw, with_deg=True)
    deg3 = deg.reshape(NW, NP, 1)
    table2 = _combine(acc1, deg3, w1t, b1r, c_arr, last=False)
    acc2 = _sc_aggregate(table2, src3, dst3, zeros, gpw, with_deg=False)
    out = _combine(acc2, deg3, w2t, b2r, c_arr, last=True)
    return out[:N]
